# Initial kernel scaffold; baseline (speedup 1.0000x reference)
import jax
import jax.numpy as jnp
from jax.experimental import pallas as pl


def kernel(from_vertices, to_vertices):
    raise NotImplementedError("write your pallas kernel here")


# speedup vs baseline: 4.8837x; 4.8837x over previous
import functools

import jax
import jax.numpy as jnp
from jax.experimental import pallas as pl

MAX_ITERATIONS = 100
RELATIVE_RMSE_THR = 1e-05
_QB = 512


def _knn_kernel(xt_ref, y_ref, ytr_ref, ynn_ref, *, n_keys):
    xt = xt_ref[0]
    y = y_ref[0]
    ytr = ytr_ref[0]

    mm = jax.lax.dot_general(xt, y, (((1,), (1,)), ((), ())),
                             preferred_element_type=jnp.float32)
    xx = (xt[:, 0:1] * xt[:, 0:1] + xt[:, 1:2] * xt[:, 1:2]
          + xt[:, 2:3] * xt[:, 2:3])
    d = (xx + ytr[3:4, :]) - 2.0 * mm

    minv = jnp.min(d, axis=1, keepdims=True)
    iota = jax.lax.broadcasted_iota(jnp.int32, d.shape, 1)
    idx = jnp.min(jnp.where(d == minv, iota, n_keys),
                  axis=1, keepdims=True)
    mask = iota == idx

    zero = jnp.zeros_like(d)
    ynn_ref[0] = jnp.concatenate(
        [jnp.sum(jnp.where(mask, ytr[j:j + 1, :], zero), axis=1, keepdims=True)
         for j in range(3)], axis=1)


def _alignment(x, y):
    b, n, dim = x.shape
    xmu = jnp.mean(x, axis=1, keepdims=True)
    ymu = jnp.mean(y, axis=1, keepdims=True)
    xc = x - xmu
    yc = y - ymu
    xycov = jnp.einsum('bni,bnj->bij', xc, yc) / n
    u, _, vh = jnp.linalg.svd(xycov, full_matrices=False)
    detuv = jnp.linalg.det(jnp.matmul(u, vh))
    e = jnp.tile(jnp.eye(dim, dtype=x.dtype)[None], (b, 1, 1))
    e = e.at[:, -1, -1].set(detuv)
    r = jnp.matmul(u, jnp.matmul(e, vh))
    t = ymu[:, 0, :] - jnp.matmul(xmu, r)[:, 0, :]
    return r, t


def kernel(from_vertices, to_vertices):
    x = from_vertices
    y = to_vertices
    b, n, dim = x.shape
    f32 = jnp.float32

    knn = pl.pallas_call(
        functools.partial(_knn_kernel, n_keys=n),
        grid=(b, n // _QB),
        in_specs=[
            pl.BlockSpec((1, _QB, dim), lambda i, j: (i, j, 0)),
            pl.BlockSpec((1, n, dim), lambda i, j: (i, 0, 0)),
            pl.BlockSpec((1, 4, n), lambda i, j: (i, 0, 0)),
        ],
        out_specs=pl.BlockSpec((1, _QB, dim), lambda i, j: (i, j, 0)),
        out_shape=jax.ShapeDtypeStruct((b, n, dim), f32),
    )

    r0, t0 = _alignment(x, y)
    fv = jnp.matmul(x, r0) + t0[:, None, :]

    ysq = jnp.sum(y * y, axis=-1)
    ytr = jnp.concatenate([jnp.swapaxes(y, 1, 2), ysq[:, None, :]], axis=1)

    def cond(carry):
        _, _, converged, i = carry
        return jnp.logical_and(jnp.logical_not(converged), i < MAX_ITERATIONS)

    def body(carry):
        xt, prev_rmse, _, i = carry
        ynn = knn(xt, y, ytr)
        r, t = _alignment(fv, ynn)
        xt_new = jnp.matmul(fv, r) + t[:, None, :]
        sq = jnp.sum((xt_new - ynn) ** 2, axis=-1)
        rmse = jnp.sqrt(jnp.mean(sq, axis=1))
        rel = jnp.where(i == 0, jnp.ones((b,), dtype=rmse.dtype),
                        (prev_rmse - rmse) / prev_rmse)
        newly = jnp.all(rel <= RELATIVE_RMSE_THR)
        return (xt_new, rmse, newly, i + 1)

    carry0 = (fv, jnp.ones((b,), f32), jnp.array(False), jnp.int32(0))
    xt_fin, _, _, _ = jax.lax.while_loop(cond, body, carry0)
    return xt_fin
